# Initial kernel scaffold; baseline (speedup 1.0000x reference)
import jax
import jax.numpy as jnp
from jax.experimental import pallas as pl


def kernel(inputs, g1, Wq, Wk, Wv, Wo, g2, Wg, W1, b1, W2, b2):
    raise NotImplementedError("write your pallas kernel here")


# speedup vs baseline: 1.3405x; 1.3405x over previous
import jax
import jax.numpy as jnp
from jax.experimental import pallas as pl
from jax.experimental.pallas import tpu as pltpu

B, S, D = 1, 2048, 1024
H = 16
DH = D // H
F = 2048
E = 8
EPS = 1e-6
TS = 512
NT = S // TS


def _qkv_body(x_ref, g1_ref, wq_ref, wk_ref, wv_ref, q_ref, k_ref, v_ref):
    x = x_ref[...]
    ms = jnp.mean(jnp.square(x), axis=-1, keepdims=True)
    xn = (x * jax.lax.rsqrt(ms + EPS) * g1_ref[...]).astype(jnp.bfloat16)
    q = jnp.dot(xn, wq_ref[...], preferred_element_type=jnp.float32)
    q_ref[...] = (q * (1.0 / (DH ** 0.5))).astype(jnp.bfloat16)
    k_ref[...] = jnp.dot(xn, wk_ref[...],
                         preferred_element_type=jnp.float32).astype(jnp.bfloat16)
    v_ref[...] = jnp.dot(xn, wv_ref[...],
                         preferred_element_type=jnp.float32).astype(jnp.bfloat16)


def _attn_body(q_ref, k_ref, v_ref, o_ref):
    q = q_ref[0]
    k = k_ref[0]
    s = jax.lax.dot_general(q, k, (((1,), (1,)), ((), ())),
                            preferred_element_type=jnp.float32)
    m = jnp.max(s, axis=-1, keepdims=True)
    p = jnp.exp(s - m)
    p = p / jnp.sum(p, axis=-1, keepdims=True)
    o_ref[0] = jnp.dot(p.astype(jnp.bfloat16), v_ref[0],
                       preferred_element_type=jnp.float32).astype(jnp.bfloat16)


def _post_body(o_ref, wo_ref, inp_ref, g2_ref, x1_ref, xn2_ref):
    o = jnp.dot(o_ref[...], wo_ref[...], preferred_element_type=jnp.float32)
    x1 = o + inp_ref[...]
    x1_ref[...] = x1
    ms = jnp.mean(jnp.square(x1), axis=-1, keepdims=True)
    xn2_ref[...] = (x1 * jax.lax.rsqrt(ms + EPS) * g2_ref[...]).astype(jnp.bfloat16)


def _moe_body(xn_ref, x1_ref, wg_ref, w1_ref, b1_ref, w2_ref, b2_ref, out_ref):
    e = pl.program_id(1)
    xn = xn_ref[...]
    logits = jnp.dot(xn, wg_ref[...], preferred_element_type=jnp.float32)
    m = jnp.max(logits, axis=-1, keepdims=True)
    p = jnp.exp(logits - m)
    gate = p / jnp.sum(p, axis=-1, keepdims=True)

    @pl.when(e == 0)
    def _init():
        out_ref[...] = x1_ref[...] + jnp.dot(
            gate, b2_ref[...], preferred_element_type=jnp.float32)

    h = jnp.dot(xn, w1_ref[0], preferred_element_type=jnp.float32) + b1_ref[0]
    h = jnp.maximum(h, 0.0)
    cols = jax.lax.broadcasted_iota(jnp.int32, (TS, E), 1)
    ge = jnp.sum(jnp.where(cols == e, gate, 0.0), axis=-1, keepdims=True)
    h = (h * ge).astype(jnp.bfloat16)
    out_ref[...] += jnp.dot(h, w2_ref[0], preferred_element_type=jnp.float32)


def kernel(inputs, g1, Wq, Wk, Wv, Wo, g2, Wg, W1, b1, W2, b2):
    x = inputs.reshape(S, D)
    g1r = g1.reshape(1, D)
    g2r = g2.reshape(1, D)
    wq = Wq.astype(jnp.bfloat16)
    wk = Wk.astype(jnp.bfloat16)
    wv = Wv.astype(jnp.bfloat16)
    wo = Wo.astype(jnp.bfloat16)
    w1 = W1.astype(jnp.bfloat16)
    w2 = W2.astype(jnp.bfloat16)

    full = lambda shp: pl.BlockSpec(shp, lambda *_: tuple(0 for _ in shp))
    tok = pl.BlockSpec((TS, D), lambda t: (t, 0))

    q, k, v = pl.pallas_call(
        _qkv_body,
        grid=(NT,),
        in_specs=[tok, full((1, D)), full((D, D)), full((D, D)), full((D, D))],
        out_specs=[tok, tok, tok],
        out_shape=[jax.ShapeDtypeStruct((S, D), jnp.bfloat16)] * 3,
        compiler_params=pltpu.CompilerParams(
            dimension_semantics=("arbitrary",)),
    )(x, g1r, wq, wk, wv)

    qh = q.reshape(S, H, DH).transpose(1, 0, 2)
    kh = k.reshape(S, H, DH).transpose(1, 0, 2)
    vh = v.reshape(S, H, DH).transpose(1, 0, 2)

    head = pl.BlockSpec((1, S, DH), lambda h: (h, 0, 0))
    oh = pl.pallas_call(
        _attn_body,
        grid=(H,),
        in_specs=[head, head, head],
        out_specs=head,
        out_shape=jax.ShapeDtypeStruct((H, S, DH), jnp.bfloat16),
        compiler_params=pltpu.CompilerParams(
            dimension_semantics=("arbitrary",)),
    )(qh, kh, vh)

    o = oh.transpose(1, 0, 2).reshape(S, D)

    x1, xn2 = pl.pallas_call(
        _post_body,
        grid=(NT,),
        in_specs=[tok, full((D, D)), tok, full((1, D))],
        out_specs=[tok, tok],
        out_shape=[jax.ShapeDtypeStruct((S, D), jnp.float32),
                   jax.ShapeDtypeStruct((S, D), jnp.bfloat16)],
        compiler_params=pltpu.CompilerParams(
            dimension_semantics=("arbitrary",)),
    )(o, wo, x, g2r)

    out = pl.pallas_call(
        _moe_body,
        grid=(NT, E),
        in_specs=[
            pl.BlockSpec((TS, D), lambda t, e: (t, 0)),
            pl.BlockSpec((TS, D), lambda t, e: (t, 0)),
            pl.BlockSpec((D, E), lambda t, e: (0, 0)),
            pl.BlockSpec((1, D, F), lambda t, e: (e, 0, 0)),
            pl.BlockSpec((1, 1, F), lambda t, e: (e, 0, 0)),
            pl.BlockSpec((1, F, D), lambda t, e: (e, 0, 0)),
            pl.BlockSpec((E, D), lambda t, e: (0, 0)),
        ],
        out_specs=pl.BlockSpec((TS, D), lambda t, e: (t, 0)),
        out_shape=jax.ShapeDtypeStruct((S, D), jnp.float32),
        compiler_params=pltpu.CompilerParams(
            dimension_semantics=("parallel", "arbitrary")),
    )(xn2, x1, Wg, w1, b1.reshape(E, 1, F), w2, b2)

    return out.reshape(B, S, D)


# speedup vs baseline: 1.5548x; 1.1598x over previous
import jax
import jax.numpy as jnp
from jax.experimental import pallas as pl
from jax.experimental.pallas import tpu as pltpu

B, S, D = 1, 2048, 1024
H = 16
DH = D // H
F = 2048
E = 8
EPS = 1e-6
TS = 512
NT = S // TS


def _qkv_body(x_ref, g1_ref, wq_ref, wk_ref, wv_ref, q_ref, k_ref, v_ref):
    x = x_ref[...]
    ms = jnp.mean(jnp.square(x), axis=-1, keepdims=True)
    xn = (x * jax.lax.rsqrt(ms + EPS) * g1_ref[...]).astype(jnp.bfloat16)
    q = jnp.dot(xn, wq_ref[...], preferred_element_type=jnp.float32)
    q_ref[...] = (q * (1.0 / (DH ** 0.5))).astype(jnp.bfloat16)
    k_ref[...] = jnp.dot(xn, wk_ref[...],
                         preferred_element_type=jnp.float32).astype(jnp.bfloat16)
    v_ref[...] = jnp.dot(xn, wv_ref[...],
                         preferred_element_type=jnp.float32).astype(jnp.bfloat16)


CS = 512


def _attn_body(q_ref, k_ref, v_ref, o_ref):
    k = k_ref[0]
    v = v_ref[0]
    for j in range(S // CS):
        q = q_ref[0, pl.ds(j * CS, CS), :]
        s = jax.lax.dot_general(q, k, (((1,), (1,)), ((), ())),
                                preferred_element_type=jnp.float32)
        m = jnp.max(s, axis=-1, keepdims=True)
        p = jnp.exp(s - m)
        r = jnp.sum(p, axis=-1, keepdims=True)
        o = jnp.dot(p.astype(jnp.bfloat16), v,
                    preferred_element_type=jnp.float32)
        o_ref[0, pl.ds(j * CS, CS), :] = (o / r).astype(jnp.bfloat16)


def _post_body(o_ref, wo_ref, inp_ref, g2_ref, x1_ref, xn2_ref):
    o = jnp.dot(o_ref[...], wo_ref[...], preferred_element_type=jnp.float32)
    x1 = o + inp_ref[...]
    x1_ref[...] = x1
    ms = jnp.mean(jnp.square(x1), axis=-1, keepdims=True)
    xn2_ref[...] = (x1 * jax.lax.rsqrt(ms + EPS) * g2_ref[...]).astype(jnp.bfloat16)


def _moe_body(xn_ref, x1_ref, wg_ref, w1_ref, b1_ref, w2_ref, b2_ref, out_ref):
    e = pl.program_id(1)
    xn = xn_ref[...]
    logits = jnp.dot(xn, wg_ref[...], preferred_element_type=jnp.float32)
    m = jnp.max(logits, axis=-1, keepdims=True)
    p = jnp.exp(logits - m)
    gate = p / jnp.sum(p, axis=-1, keepdims=True)

    @pl.when(e == 0)
    def _init():
        out_ref[...] = x1_ref[...] + jnp.dot(
            gate, b2_ref[...], preferred_element_type=jnp.float32)

    h = jnp.dot(xn, w1_ref[0], preferred_element_type=jnp.float32) + b1_ref[0]
    h = jnp.maximum(h, 0.0)
    cols = jax.lax.broadcasted_iota(jnp.int32, (TS, E), 1)
    ge = jnp.sum(jnp.where(cols == e, gate, 0.0), axis=-1, keepdims=True)
    h = (h * ge).astype(jnp.bfloat16)
    out_ref[...] += jnp.dot(h, w2_ref[0], preferred_element_type=jnp.float32)


def kernel(inputs, g1, Wq, Wk, Wv, Wo, g2, Wg, W1, b1, W2, b2):
    x = inputs.reshape(S, D)
    g1r = g1.reshape(1, D)
    g2r = g2.reshape(1, D)
    wq = Wq.astype(jnp.bfloat16)
    wk = Wk.astype(jnp.bfloat16)
    wv = Wv.astype(jnp.bfloat16)
    wo = Wo.astype(jnp.bfloat16)
    w1 = W1.astype(jnp.bfloat16)
    w2 = W2.astype(jnp.bfloat16)

    full = lambda shp: pl.BlockSpec(shp, lambda *_: tuple(0 for _ in shp))
    tok = pl.BlockSpec((TS, D), lambda t: (t, 0))

    q, k, v = pl.pallas_call(
        _qkv_body,
        grid=(NT,),
        in_specs=[tok, full((1, D)), full((D, D)), full((D, D)), full((D, D))],
        out_specs=[tok, tok, tok],
        out_shape=[jax.ShapeDtypeStruct((S, D), jnp.bfloat16)] * 3,
        compiler_params=pltpu.CompilerParams(
            dimension_semantics=("arbitrary",)),
    )(x, g1r, wq, wk, wv)

    qh = q.reshape(S, H, DH).transpose(1, 0, 2)
    kh = k.reshape(S, H, DH).transpose(1, 0, 2)
    vh = v.reshape(S, H, DH).transpose(1, 0, 2)

    head = pl.BlockSpec((1, S, DH), lambda h: (h, 0, 0))
    oh = pl.pallas_call(
        _attn_body,
        grid=(H,),
        in_specs=[head, head, head],
        out_specs=head,
        out_shape=jax.ShapeDtypeStruct((H, S, DH), jnp.bfloat16),
        compiler_params=pltpu.CompilerParams(
            dimension_semantics=("arbitrary",)),
    )(qh, kh, vh)

    o = oh.transpose(1, 0, 2).reshape(S, D)

    x1, xn2 = pl.pallas_call(
        _post_body,
        grid=(NT,),
        in_specs=[tok, full((D, D)), tok, full((1, D))],
        out_specs=[tok, tok],
        out_shape=[jax.ShapeDtypeStruct((S, D), jnp.float32),
                   jax.ShapeDtypeStruct((S, D), jnp.bfloat16)],
        compiler_params=pltpu.CompilerParams(
            dimension_semantics=("arbitrary",)),
    )(o, wo, x, g2r)

    out = pl.pallas_call(
        _moe_body,
        grid=(NT, E),
        in_specs=[
            pl.BlockSpec((TS, D), lambda t, e: (t, 0)),
            pl.BlockSpec((TS, D), lambda t, e: (t, 0)),
            pl.BlockSpec((D, E), lambda t, e: (0, 0)),
            pl.BlockSpec((1, D, F), lambda t, e: (e, 0, 0)),
            pl.BlockSpec((1, 1, F), lambda t, e: (e, 0, 0)),
            pl.BlockSpec((1, F, D), lambda t, e: (e, 0, 0)),
            pl.BlockSpec((E, D), lambda t, e: (0, 0)),
        ],
        out_specs=pl.BlockSpec((TS, D), lambda t, e: (t, 0)),
        out_shape=jax.ShapeDtypeStruct((S, D), jnp.float32),
        compiler_params=pltpu.CompilerParams(
            dimension_semantics=("parallel", "arbitrary")),
    )(xn2, x1, Wg, w1, b1.reshape(E, 1, F), w2, b2)

    return out.reshape(B, S, D)


# speedup vs baseline: 1.5928x; 1.0245x over previous
import jax
import jax.numpy as jnp
from jax.experimental import pallas as pl
from jax.experimental.pallas import tpu as pltpu

B, S, D = 1, 2048, 1024
H = 16
DH = D // H
F = 2048
E = 8
EPS = 1e-6
TS = 512
NT = S // TS
MTS = 1024
MNT = S // MTS


def _qkv_body(x_ref, g1_ref, wq_ref, wk_ref, wv_ref, q_ref, k_ref, v_ref):
    x = x_ref[...]
    ms = jnp.mean(jnp.square(x), axis=-1, keepdims=True)
    xn = (x * jax.lax.rsqrt(ms + EPS) * g1_ref[...]).astype(jnp.bfloat16)
    q = jnp.dot(xn, wq_ref[...], preferred_element_type=jnp.float32)
    q_ref[...] = (q * (1.0 / (DH ** 0.5))).astype(jnp.bfloat16)
    k_ref[...] = jnp.dot(xn, wk_ref[...],
                         preferred_element_type=jnp.float32).astype(jnp.bfloat16)
    v_ref[...] = jnp.dot(xn, wv_ref[...],
                         preferred_element_type=jnp.float32).astype(jnp.bfloat16)


CS = 512


def _attn_body(q_ref, k_ref, v_ref, o_ref):
    k = k_ref[0]
    v = v_ref[0]
    for j in range(S // CS):
        q = q_ref[0, pl.ds(j * CS, CS), :]
        s = jax.lax.dot_general(q, k, (((1,), (1,)), ((), ())),
                                preferred_element_type=jnp.float32)
        m = jnp.max(s, axis=-1, keepdims=True)
        p = jnp.exp(s - m)
        r = jnp.sum(p, axis=-1, keepdims=True)
        o = jnp.dot(p.astype(jnp.bfloat16), v,
                    preferred_element_type=jnp.float32)
        o_ref[0, pl.ds(j * CS, CS), :] = (o / r).astype(jnp.bfloat16)


def _post_body(o_ref, wo_ref, inp_ref, g2_ref, x1_ref, xn2_ref):
    o = jnp.dot(o_ref[...], wo_ref[...], preferred_element_type=jnp.float32)
    x1 = o + inp_ref[...]
    x1_ref[...] = x1
    ms = jnp.mean(jnp.square(x1), axis=-1, keepdims=True)
    xn2_ref[...] = (x1 * jax.lax.rsqrt(ms + EPS) * g2_ref[...]).astype(jnp.bfloat16)


def _moe_body(xn_ref, x1_ref, wg_ref, w1_ref, b1_ref, w2_ref, b2_ref, out_ref):
    e = pl.program_id(1)
    xn = xn_ref[...]
    logits = jnp.dot(xn, wg_ref[...], preferred_element_type=jnp.float32)
    m = jnp.max(logits, axis=-1, keepdims=True)
    p = jnp.exp(logits - m)
    gate = p / jnp.sum(p, axis=-1, keepdims=True)

    @pl.when(e == 0)
    def _init():
        out_ref[...] = x1_ref[...] + jnp.dot(
            gate, b2_ref[...], preferred_element_type=jnp.float32)

    h = jnp.dot(xn, w1_ref[0], preferred_element_type=jnp.float32) + b1_ref[0]
    h = jnp.maximum(h, 0.0)
    cols = jax.lax.broadcasted_iota(jnp.int32, (MTS, E), 1)
    ge = jnp.sum(jnp.where(cols == e, gate, 0.0), axis=-1, keepdims=True)
    h = (h * ge).astype(jnp.bfloat16)
    out_ref[...] += jnp.dot(h, w2_ref[0], preferred_element_type=jnp.float32)


def kernel(inputs, g1, Wq, Wk, Wv, Wo, g2, Wg, W1, b1, W2, b2):
    x = inputs.reshape(S, D)
    g1r = g1.reshape(1, D)
    g2r = g2.reshape(1, D)
    wq = Wq.astype(jnp.bfloat16)
    wk = Wk.astype(jnp.bfloat16)
    wv = Wv.astype(jnp.bfloat16)
    wo = Wo.astype(jnp.bfloat16)
    w1 = W1.astype(jnp.bfloat16)
    w2 = W2.astype(jnp.bfloat16)

    full = lambda shp: pl.BlockSpec(shp, lambda *_: tuple(0 for _ in shp))
    tok = pl.BlockSpec((TS, D), lambda t: (t, 0))

    q, k, v = pl.pallas_call(
        _qkv_body,
        grid=(NT,),
        in_specs=[tok, full((1, D)), full((D, D)), full((D, D)), full((D, D))],
        out_specs=[tok, tok, tok],
        out_shape=[jax.ShapeDtypeStruct((S, D), jnp.bfloat16)] * 3,
        compiler_params=pltpu.CompilerParams(
            dimension_semantics=("arbitrary",)),
    )(x, g1r, wq, wk, wv)

    qh = q.reshape(S, H, DH).transpose(1, 0, 2)
    kh = k.reshape(S, H, DH).transpose(1, 0, 2)
    vh = v.reshape(S, H, DH).transpose(1, 0, 2)

    head = pl.BlockSpec((1, S, DH), lambda h: (h, 0, 0))
    oh = pl.pallas_call(
        _attn_body,
        grid=(H,),
        in_specs=[head, head, head],
        out_specs=head,
        out_shape=jax.ShapeDtypeStruct((H, S, DH), jnp.bfloat16),
        compiler_params=pltpu.CompilerParams(
            dimension_semantics=("arbitrary",)),
    )(qh, kh, vh)

    o = oh.transpose(1, 0, 2).reshape(S, D)

    x1, xn2 = pl.pallas_call(
        _post_body,
        grid=(NT,),
        in_specs=[tok, full((D, D)), tok, full((1, D))],
        out_specs=[tok, tok],
        out_shape=[jax.ShapeDtypeStruct((S, D), jnp.float32),
                   jax.ShapeDtypeStruct((S, D), jnp.bfloat16)],
        compiler_params=pltpu.CompilerParams(
            dimension_semantics=("arbitrary",)),
    )(o, wo, x, g2r)

    out = pl.pallas_call(
        _moe_body,
        grid=(MNT, E),
        in_specs=[
            pl.BlockSpec((MTS, D), lambda t, e: (t, 0)),
            pl.BlockSpec((MTS, D), lambda t, e: (t, 0)),
            pl.BlockSpec((D, E), lambda t, e: (0, 0)),
            pl.BlockSpec((1, D, F), lambda t, e: (e, 0, 0)),
            pl.BlockSpec((1, 1, F), lambda t, e: (e, 0, 0)),
            pl.BlockSpec((1, F, D), lambda t, e: (e, 0, 0)),
            pl.BlockSpec((E, D), lambda t, e: (0, 0)),
        ],
        out_specs=pl.BlockSpec((MTS, D), lambda t, e: (t, 0)),
        out_shape=jax.ShapeDtypeStruct((S, D), jnp.float32),
        compiler_params=pltpu.CompilerParams(
            dimension_semantics=("parallel", "arbitrary")),
    )(xn2, x1, Wg, w1, b1.reshape(E, 1, F), w2, b2)

    return out.reshape(B, S, D)


# speedup vs baseline: 1.7119x; 1.0748x over previous
import jax
import jax.numpy as jnp
from jax.experimental import pallas as pl
from jax.experimental.pallas import tpu as pltpu

B, S, D = 1, 2048, 1024
H = 16
DH = D // H
F = 2048
E = 8
EPS = 1e-6
TS = 512
NT = S // TS
MTS = 1024
MNT = S // MTS


def _qkv_body(x_ref, g1_ref, wq_ref, wk_ref, wv_ref, q_ref, k_ref, v_ref):
    x = x_ref[...]
    ms = jnp.mean(jnp.square(x), axis=-1, keepdims=True)
    xn = (x * jax.lax.rsqrt(ms + EPS) * g1_ref[...]).astype(jnp.bfloat16)
    q = jnp.dot(xn, wq_ref[...], preferred_element_type=jnp.float32)
    q_ref[...] = (q * (1.0 / (DH ** 0.5))).astype(
        jnp.bfloat16).reshape(TS, H, DH).swapaxes(0, 1)
    k_ref[...] = jnp.dot(xn, wk_ref[...],
                         preferred_element_type=jnp.float32).astype(
        jnp.bfloat16).reshape(TS, H, DH).swapaxes(0, 1)
    v_ref[...] = jnp.dot(xn, wv_ref[...],
                         preferred_element_type=jnp.float32).astype(
        jnp.bfloat16).reshape(TS, H, DH).swapaxes(0, 1)


CS = 512


def _attn_body(q_ref, k_ref, v_ref, o_ref):
    k = k_ref[0]
    v = v_ref[0]
    for j in range(S // CS):
        q = q_ref[0, pl.ds(j * CS, CS), :]
        s = jax.lax.dot_general(q, k, (((1,), (1,)), ((), ())),
                                preferred_element_type=jnp.float32)
        m = jnp.max(s, axis=-1, keepdims=True)
        p = jnp.exp(s - m)
        r = jnp.sum(p, axis=-1, keepdims=True)
        o = jnp.dot(p.astype(jnp.bfloat16), v,
                    preferred_element_type=jnp.float32)
        o_ref[0, pl.ds(j * CS, CS), :] = (o / r).astype(jnp.bfloat16)


def _post_body(o_ref, wo_ref, inp_ref, g2_ref, x1_ref, xn2_ref):
    oh = o_ref[...].swapaxes(0, 1).reshape(TS, D)
    o = jnp.dot(oh, wo_ref[...], preferred_element_type=jnp.float32)
    x1 = o + inp_ref[...]
    x1_ref[...] = x1
    ms = jnp.mean(jnp.square(x1), axis=-1, keepdims=True)
    xn2_ref[...] = (x1 * jax.lax.rsqrt(ms + EPS) * g2_ref[...]).astype(jnp.bfloat16)


def _moe_body(xn_ref, x1_ref, wg_ref, w1_ref, b1_ref, w2_ref, b2_ref, out_ref):
    e = pl.program_id(1)
    xn = xn_ref[...]
    logits = jnp.dot(xn, wg_ref[...], preferred_element_type=jnp.float32)
    m = jnp.max(logits, axis=-1, keepdims=True)
    p = jnp.exp(logits - m)
    gate = p / jnp.sum(p, axis=-1, keepdims=True)

    @pl.when(e == 0)
    def _init():
        out_ref[...] = x1_ref[...] + jnp.dot(
            gate, b2_ref[...], preferred_element_type=jnp.float32)

    h = jnp.dot(xn, w1_ref[0], preferred_element_type=jnp.float32) + b1_ref[0]
    h = jnp.maximum(h, 0.0)
    cols = jax.lax.broadcasted_iota(jnp.int32, (MTS, E), 1)
    ge = jnp.sum(jnp.where(cols == e, gate, 0.0), axis=-1, keepdims=True)
    h = (h * ge).astype(jnp.bfloat16)
    out_ref[...] += jnp.dot(h, w2_ref[0], preferred_element_type=jnp.float32)


def kernel(inputs, g1, Wq, Wk, Wv, Wo, g2, Wg, W1, b1, W2, b2):
    x = inputs.reshape(S, D)
    g1r = g1.reshape(1, D)
    g2r = g2.reshape(1, D)
    wq = Wq.astype(jnp.bfloat16)
    wk = Wk.astype(jnp.bfloat16)
    wv = Wv.astype(jnp.bfloat16)
    wo = Wo.astype(jnp.bfloat16)
    w1 = W1.astype(jnp.bfloat16)
    w2 = W2.astype(jnp.bfloat16)

    full = lambda shp: pl.BlockSpec(shp, lambda *_: tuple(0 for _ in shp))
    tok = pl.BlockSpec((TS, D), lambda t: (t, 0))
    tokh = pl.BlockSpec((H, TS, DH), lambda t: (0, t, 0))

    qh, kh, vh = pl.pallas_call(
        _qkv_body,
        grid=(NT,),
        in_specs=[tok, full((1, D)), full((D, D)), full((D, D)), full((D, D))],
        out_specs=[tokh, tokh, tokh],
        out_shape=[jax.ShapeDtypeStruct((H, S, DH), jnp.bfloat16)] * 3,
        compiler_params=pltpu.CompilerParams(
            dimension_semantics=("arbitrary",)),
    )(x, g1r, wq, wk, wv)

    head = pl.BlockSpec((1, S, DH), lambda h: (h, 0, 0))
    oh = pl.pallas_call(
        _attn_body,
        grid=(H,),
        in_specs=[head, head, head],
        out_specs=head,
        out_shape=jax.ShapeDtypeStruct((H, S, DH), jnp.bfloat16),
        compiler_params=pltpu.CompilerParams(
            dimension_semantics=("arbitrary",)),
    )(qh, kh, vh)

    x1, xn2 = pl.pallas_call(
        _post_body,
        grid=(NT,),
        in_specs=[tokh, full((D, D)), tok, full((1, D))],
        out_specs=[tok, tok],
        out_shape=[jax.ShapeDtypeStruct((S, D), jnp.float32),
                   jax.ShapeDtypeStruct((S, D), jnp.bfloat16)],
        compiler_params=pltpu.CompilerParams(
            dimension_semantics=("arbitrary",)),
    )(oh, wo, x, g2r)

    out = pl.pallas_call(
        _moe_body,
        grid=(MNT, E),
        in_specs=[
            pl.BlockSpec((MTS, D), lambda t, e: (t, 0)),
            pl.BlockSpec((MTS, D), lambda t, e: (t, 0)),
            pl.BlockSpec((D, E), lambda t, e: (0, 0)),
            pl.BlockSpec((1, D, F), lambda t, e: (e, 0, 0)),
            pl.BlockSpec((1, 1, F), lambda t, e: (e, 0, 0)),
            pl.BlockSpec((1, F, D), lambda t, e: (e, 0, 0)),
            pl.BlockSpec((E, D), lambda t, e: (0, 0)),
        ],
        out_specs=pl.BlockSpec((MTS, D), lambda t, e: (t, 0)),
        out_shape=jax.ShapeDtypeStruct((S, D), jnp.float32),
        compiler_params=pltpu.CompilerParams(
            dimension_semantics=("parallel", "arbitrary")),
    )(xn2, x1, Wg, w1, b1.reshape(E, 1, F), w2, b2)

    return out.reshape(B, S, D)


# speedup vs baseline: 1.9289x; 1.1267x over previous
import jax
import jax.numpy as jnp
from jax.experimental import pallas as pl
from jax.experimental.pallas import tpu as pltpu

B, S, D = 1, 2048, 1024
H = 16
DH = D // H
F = 2048
E = 8
EPS = 1e-6
TS = 512
NT = S // TS
MTS = 1024
MNT = S // MTS


def _qkv_body(x_ref, g1_ref, wq_ref, wk_ref, wv_ref, q_ref, k_ref, v_ref):
    x = x_ref[...]
    ms = jnp.mean(jnp.square(x), axis=-1, keepdims=True)
    xn = (x * jax.lax.rsqrt(ms + EPS) * g1_ref[...]).astype(jnp.bfloat16)
    q = jnp.dot(xn, wq_ref[...], preferred_element_type=jnp.float32)
    q_ref[...] = (q * (1.0 / (DH ** 0.5))).astype(
        jnp.bfloat16).reshape(TS, H, DH).swapaxes(0, 1)
    k_ref[...] = jnp.dot(xn, wk_ref[...],
                         preferred_element_type=jnp.float32).astype(
        jnp.bfloat16).reshape(TS, H, DH).swapaxes(0, 1)
    v_ref[...] = jnp.dot(xn, wv_ref[...],
                         preferred_element_type=jnp.float32).astype(
        jnp.bfloat16).reshape(TS, H, DH).swapaxes(0, 1)


CS = 512


def _attn_body(q_ref, k_ref, v_ref, o_ref):
    k = k_ref[0]
    v = v_ref[0]
    for j in range(S // CS):
        q = q_ref[0, pl.ds(j * CS, CS), :]
        s = jax.lax.dot_general(q, k, (((1,), (1,)), ((), ())),
                                preferred_element_type=jnp.float32)
        m = jnp.max(s, axis=-1, keepdims=True)
        p = jnp.exp(s - m)
        r = jnp.sum(p, axis=-1, keepdims=True)
        o = jnp.dot(p.astype(jnp.bfloat16), v,
                    preferred_element_type=jnp.float32)
        o_ref[0, pl.ds(j * CS, CS), :] = (o / r).astype(jnp.bfloat16)


def _post_body(o_ref, wo_ref, inp_ref, g2_ref, x1_ref, xn2_ref):
    oh = o_ref[...].swapaxes(0, 1).reshape(TS, D)
    o = jnp.dot(oh, wo_ref[...], preferred_element_type=jnp.float32)
    x1 = o + inp_ref[...]
    x1_ref[...] = x1
    ms = jnp.mean(jnp.square(x1), axis=-1, keepdims=True)
    xn2_ref[...] = (x1 * jax.lax.rsqrt(ms + EPS) * g2_ref[...]).astype(jnp.bfloat16)


FH = 2
FB = F // FH
MC = 1024


def _moe_body(xn_ref, x1_ref, wg_ref, w1_ref, b1_ref, w2_ref, b2_ref, out_ref):
    e = pl.program_id(0)
    fh = pl.program_id(1)
    first = (e == 0) & (fh == 0)
    w1b = w1_ref[0].astype(jnp.bfloat16)
    w2b = w2_ref[0].astype(jnp.bfloat16)
    b1v = b1_ref[0]
    for j in range(S // MC):
        sl = pl.ds(j * MC, MC)
        xn = xn_ref[sl, :]
        logits = jnp.dot(xn, wg_ref[...], preferred_element_type=jnp.float32)
        m = jnp.max(logits, axis=-1, keepdims=True)
        p = jnp.exp(logits - m)
        gate = p / jnp.sum(p, axis=-1, keepdims=True)
        cols = jax.lax.broadcasted_iota(jnp.int32, (MC, E), 1)
        ge = jnp.sum(jnp.where(cols == e, gate, 0.0), axis=-1, keepdims=True)
        h = jnp.dot(xn, w1b, preferred_element_type=jnp.float32) + b1v
        h = (jnp.maximum(h, 0.0) * ge).astype(jnp.bfloat16)
        contrib = jnp.dot(h, w2b, preferred_element_type=jnp.float32)

        @pl.when(first)
        def _init():
            out_ref[sl, :] = x1_ref[sl, :] + contrib + jnp.dot(
                gate, b2_ref[...], preferred_element_type=jnp.float32)

        @pl.when(jnp.logical_not(first))
        def _acc():
            out_ref[sl, :] += contrib


def kernel(inputs, g1, Wq, Wk, Wv, Wo, g2, Wg, W1, b1, W2, b2):
    x = inputs.reshape(S, D)
    g1r = g1.reshape(1, D)
    g2r = g2.reshape(1, D)
    wq = Wq.astype(jnp.bfloat16)
    wk = Wk.astype(jnp.bfloat16)
    wv = Wv.astype(jnp.bfloat16)
    wo = Wo.astype(jnp.bfloat16)

    full = lambda shp: pl.BlockSpec(shp, lambda *_: tuple(0 for _ in shp))
    tok = pl.BlockSpec((TS, D), lambda t: (t, 0))
    tokh = pl.BlockSpec((H, TS, DH), lambda t: (0, t, 0))

    qh, kh, vh = pl.pallas_call(
        _qkv_body,
        grid=(NT,),
        in_specs=[tok, full((1, D)), full((D, D)), full((D, D)), full((D, D))],
        out_specs=[tokh, tokh, tokh],
        out_shape=[jax.ShapeDtypeStruct((H, S, DH), jnp.bfloat16)] * 3,
        compiler_params=pltpu.CompilerParams(
            dimension_semantics=("arbitrary",)),
    )(x, g1r, wq, wk, wv)

    head = pl.BlockSpec((1, S, DH), lambda h: (h, 0, 0))
    oh = pl.pallas_call(
        _attn_body,
        grid=(H,),
        in_specs=[head, head, head],
        out_specs=head,
        out_shape=jax.ShapeDtypeStruct((H, S, DH), jnp.bfloat16),
        compiler_params=pltpu.CompilerParams(
            dimension_semantics=("arbitrary",)),
    )(qh, kh, vh)

    x1, xn2 = pl.pallas_call(
        _post_body,
        grid=(NT,),
        in_specs=[tokh, full((D, D)), tok, full((1, D))],
        out_specs=[tok, tok],
        out_shape=[jax.ShapeDtypeStruct((S, D), jnp.float32),
                   jax.ShapeDtypeStruct((S, D), jnp.bfloat16)],
        compiler_params=pltpu.CompilerParams(
            dimension_semantics=("arbitrary",)),
    )(oh, wo, x, g2r)

    out = pl.pallas_call(
        _moe_body,
        grid=(E, FH),
        in_specs=[
            pl.BlockSpec((S, D), lambda e, f: (0, 0)),
            pl.BlockSpec((S, D), lambda e, f: (0, 0)),
            pl.BlockSpec((D, E), lambda e, f: (0, 0)),
            pl.BlockSpec((1, D, FB), lambda e, f: (e, 0, f)),
            pl.BlockSpec((1, 1, FB), lambda e, f: (e, 0, f)),
            pl.BlockSpec((1, FB, D), lambda e, f: (e, f, 0)),
            pl.BlockSpec((E, D), lambda e, f: (0, 0)),
        ],
        out_specs=pl.BlockSpec((S, D), lambda e, f: (0, 0)),
        out_shape=jax.ShapeDtypeStruct((S, D), jnp.float32),
        compiler_params=pltpu.CompilerParams(
            dimension_semantics=("arbitrary", "arbitrary")),
    )(xn2, x1, Wg, W1, b1.reshape(E, 1, F), W2, b2)

    return out.reshape(B, S, D)


# speedup vs baseline: 2.0293x; 1.0521x over previous
import jax
import jax.numpy as jnp
from jax.experimental import pallas as pl
from jax.experimental.pallas import tpu as pltpu

B, S, D = 1, 2048, 1024
H = 16
DH = D // H
F = 2048
E = 8
EPS = 1e-6
TS = 512
NT = S // TS
MTS = 1024
MNT = S // MTS


def _qkv_body(x_ref, g1_ref, wq_ref, wk_ref, wv_ref, q_ref, k_ref, v_ref):
    x = x_ref[...]
    ms = jnp.mean(jnp.square(x), axis=-1, keepdims=True)
    xn = (x * jax.lax.rsqrt(ms + EPS) * g1_ref[...]).astype(jnp.bfloat16)
    q = jnp.dot(xn, wq_ref[...], preferred_element_type=jnp.float32)
    q_ref[...] = (q * (1.0 / (DH ** 0.5))).astype(
        jnp.bfloat16).reshape(TS, H, DH).swapaxes(0, 1)
    k_ref[...] = jnp.dot(xn, wk_ref[...],
                         preferred_element_type=jnp.float32).astype(
        jnp.bfloat16).reshape(TS, H, DH).swapaxes(0, 1)
    v_ref[...] = jnp.dot(xn, wv_ref[...],
                         preferred_element_type=jnp.float32).astype(
        jnp.bfloat16).reshape(TS, H, DH).swapaxes(0, 1)


CS = 512


def _attn_body(q_ref, k_ref, v_ref, o_ref):
    k = k_ref[0]
    v = v_ref[0]
    for j in range(S // CS):
        q = q_ref[0, pl.ds(j * CS, CS), :]
        s = jax.lax.dot_general(q, k, (((1,), (1,)), ((), ())),
                                preferred_element_type=jnp.float32)
        m = jnp.max(s, axis=-1, keepdims=True)
        p = jnp.exp(s - m)
        r = jnp.sum(p, axis=-1, keepdims=True)
        o = jnp.dot(p.astype(jnp.bfloat16), v,
                    preferred_element_type=jnp.float32)
        o_ref[0, pl.ds(j * CS, CS), :] = (o / r).astype(jnp.bfloat16)


def _post_body(o_ref, wo_ref, inp_ref, g2_ref, wg_ref, b2_ref,
               x1_ref, xn2_ref, gate_ref):
    oh = o_ref[...].swapaxes(0, 1).reshape(TS, D)
    o = jnp.dot(oh, wo_ref[...], preferred_element_type=jnp.float32)
    x1 = o + inp_ref[...]
    ms = jnp.mean(jnp.square(x1), axis=-1, keepdims=True)
    xn2 = (x1 * jax.lax.rsqrt(ms + EPS) * g2_ref[...]).astype(jnp.bfloat16)
    xn2_ref[...] = xn2
    logits = jnp.dot(xn2, wg_ref[...], preferred_element_type=jnp.float32)
    m = jnp.max(logits, axis=-1, keepdims=True)
    p = jnp.exp(logits - m)
    gate = p / jnp.sum(p, axis=-1, keepdims=True)
    gate_ref[...] = gate
    x1_ref[...] = x1 + jnp.dot(gate, b2_ref[...],
                               preferred_element_type=jnp.float32)


FH = 1
FB = F // FH
MC = 512


def _moe_body(xn_ref, x1_ref, gate_ref, w1_ref, b1_ref, w2_ref, out_ref):
    e = pl.program_id(0)
    fh = pl.program_id(1)
    first = (e == 0) & (fh == 0)
    w1b = w1_ref[0].astype(jnp.bfloat16)
    w2b = w2_ref[0].astype(jnp.bfloat16)
    b1v = b1_ref[0]
    cols = jax.lax.broadcasted_iota(jnp.int32, (MC, E), 1)
    for j in range(S // MC):
        sl = pl.ds(j * MC, MC)
        xn = xn_ref[sl, :]
        gate = gate_ref[sl, :]
        ge = jnp.sum(jnp.where(cols == e, gate, 0.0), axis=-1, keepdims=True)
        h = jnp.dot(xn, w1b, preferred_element_type=jnp.float32) + b1v
        h = (jnp.maximum(h, 0.0) * ge).astype(jnp.bfloat16)
        contrib = jnp.dot(h, w2b, preferred_element_type=jnp.float32)

        @pl.when(first)
        def _init():
            out_ref[sl, :] = x1_ref[sl, :] + contrib

        @pl.when(jnp.logical_not(first))
        def _acc():
            out_ref[sl, :] += contrib


def kernel(inputs, g1, Wq, Wk, Wv, Wo, g2, Wg, W1, b1, W2, b2):
    x = inputs.reshape(S, D)
    g1r = g1.reshape(1, D)
    g2r = g2.reshape(1, D)
    wq = Wq.astype(jnp.bfloat16)
    wk = Wk.astype(jnp.bfloat16)
    wv = Wv.astype(jnp.bfloat16)
    wo = Wo.astype(jnp.bfloat16)

    full = lambda shp: pl.BlockSpec(shp, lambda *_: tuple(0 for _ in shp))
    tok = pl.BlockSpec((TS, D), lambda t: (t, 0))
    tokh = pl.BlockSpec((H, TS, DH), lambda t: (0, t, 0))

    qh, kh, vh = pl.pallas_call(
        _qkv_body,
        grid=(NT,),
        in_specs=[tok, full((1, D)), full((D, D)), full((D, D)), full((D, D))],
        out_specs=[tokh, tokh, tokh],
        out_shape=[jax.ShapeDtypeStruct((H, S, DH), jnp.bfloat16)] * 3,
        compiler_params=pltpu.CompilerParams(
            dimension_semantics=("arbitrary",)),
    )(x, g1r, wq, wk, wv)

    head = pl.BlockSpec((1, S, DH), lambda h: (h, 0, 0))
    oh = pl.pallas_call(
        _attn_body,
        grid=(H,),
        in_specs=[head, head, head],
        out_specs=head,
        out_shape=jax.ShapeDtypeStruct((H, S, DH), jnp.bfloat16),
        compiler_params=pltpu.CompilerParams(
            dimension_semantics=("arbitrary",)),
    )(qh, kh, vh)

    x1, xn2, gate = pl.pallas_call(
        _post_body,
        grid=(NT,),
        in_specs=[tokh, full((D, D)), tok, full((1, D)),
                  full((D, E)), full((E, D))],
        out_specs=[tok, tok, pl.BlockSpec((TS, E), lambda t: (t, 0))],
        out_shape=[jax.ShapeDtypeStruct((S, D), jnp.float32),
                   jax.ShapeDtypeStruct((S, D), jnp.bfloat16),
                   jax.ShapeDtypeStruct((S, E), jnp.float32)],
        compiler_params=pltpu.CompilerParams(
            dimension_semantics=("arbitrary",)),
    )(oh, wo, x, g2r, Wg, b2)

    out = pl.pallas_call(
        _moe_body,
        grid=(E, FH),
        in_specs=[
            pl.BlockSpec((S, D), lambda e, f: (0, 0)),
            pl.BlockSpec((S, D), lambda e, f: (0, 0)),
            pl.BlockSpec((S, E), lambda e, f: (0, 0)),
            pl.BlockSpec((1, D, FB), lambda e, f: (e, 0, f)),
            pl.BlockSpec((1, 1, FB), lambda e, f: (e, 0, f)),
            pl.BlockSpec((1, FB, D), lambda e, f: (e, f, 0)),
        ],
        out_specs=pl.BlockSpec((S, D), lambda e, f: (0, 0)),
        out_shape=jax.ShapeDtypeStruct((S, D), jnp.float32),
        compiler_params=pltpu.CompilerParams(
            dimension_semantics=("arbitrary", "arbitrary"),
            vmem_limit_bytes=100 * 1024 * 1024),
    )(xn2, x1, gate, W1, b1.reshape(E, 1, F), W2)

    return out.reshape(B, S, D)
